# baseline (speedup 1.0000x reference)
import jax
import jax.numpy as jnp
from jax.experimental import pallas as pl

_N_EXPERTS = 64
_BLOCK_ROWS = 4096


def _router_body(x_ref, wt_ref, logits_ref, idx_ref, gates_ref):
    l = jnp.dot(x_ref[...], wt_ref[...], preferred_element_type=jnp.float32)
    logits_ref[...] = l
    iota = jax.lax.broadcasted_iota(jnp.int32, l.shape, 1)
    m1 = jnp.max(l, axis=-1, keepdims=True)
    i1 = jnp.min(jnp.where(l == m1, iota, _N_EXPERTS), axis=-1, keepdims=True)
    l2 = jnp.where(iota == i1, -jnp.inf, l)
    m2 = jnp.max(l2, axis=-1, keepdims=True)
    i2 = jnp.min(jnp.where(l2 == m2, iota, _N_EXPERTS), axis=-1, keepdims=True)
    idx_ref[...] = jnp.concatenate([i1, i2], axis=1)
    mm = jnp.concatenate([m1, m2], axis=1)
    ms = jnp.concatenate([m2, m1], axis=1)
    gates_ref[...] = 1.0 / (1.0 + jnp.exp(ms - mm))


@jax.jit
def kernel(x, W):
    rows, dim = x.shape
    n_experts = W.shape[0]
    wt = W.T
    grid = (rows // _BLOCK_ROWS,)
    logits, idx, gates = pl.pallas_call(
        _router_body,
        grid=grid,
        in_specs=[
            pl.BlockSpec((_BLOCK_ROWS, dim), lambda i: (i, 0)),
            pl.BlockSpec((dim, n_experts), lambda i: (0, 0)),
        ],
        out_specs=[
            pl.BlockSpec((_BLOCK_ROWS, n_experts), lambda i: (i, 0)),
            pl.BlockSpec((_BLOCK_ROWS, 2), lambda i: (i, 0)),
            pl.BlockSpec((_BLOCK_ROWS, 2), lambda i: (i, 0)),
        ],
        out_shape=[
            jax.ShapeDtypeStruct((rows, n_experts), jnp.float32),
            jax.ShapeDtypeStruct((rows, 2), jnp.int32),
            jax.ShapeDtypeStruct((rows, 2), jnp.float32),
        ],
    )(x, wt)
    return (idx, gates, logits)
